# baseline (speedup 1.0000x reference)
import jax
import jax.numpy as jnp
from jax import lax
from jax.experimental import pallas as pl
from jax.experimental.pallas import tpu as pltpu
from jax.experimental.pallas import tpu_sc as plsc

K = 64
ROWS = 128
N = 32768
LANES = 16
NVREG = N // LANES
ROWS_PER_TILE = ROWS // 32
T0 = 2.5
CAP_FAST = 320
CAP_PAD = CAP_FAST + LANES
NV_CAND = CAP_PAD // LANES
INT_MIN = -2147483648
MASK_POS = 0x7FFFFFFF


def _keys(v):
    xi = lax.bitcast_convert_type(v, jnp.int32)
    return jnp.where(xi >= 0, xi, jnp.bitwise_xor(xi, jnp.int32(MASK_POS)))


def _inv_keys(sk):
    xi = jnp.where(sk >= 0, sk, jnp.bitwise_xor(sk, jnp.int32(MASK_POS)))
    return lax.bitcast_convert_type(xi, jnp.float32)


def _search_t(count_ge, k):
    def sbody(i, p):
        bit = jnp.left_shift(jnp.int32(1), jnp.int32(31) - i)
        cand = p + bit
        return jnp.where(count_ge(cand) >= k, cand, p)
    return lax.fori_loop(0, 32, sbody, jnp.full((LANES,), INT_MIN, jnp.int32))


def _sc_body(x_hbm, out_hbm, xa_v, xb_v, cs_v, ci_v, kv_v, ki_v,
             out_v, isem, osem):
    c = lax.axis_index("c")
    s_ax = lax.axis_index("s")
    wid = s_ax * 2 + c
    lane = lax.iota(jnp.int32, LANES)
    zeros = jnp.zeros((LANES,), jnp.int32)
    sent_f = lax.bitcast_convert_type(
        jnp.full((LANES,), INT_MIN, jnp.int32), jnp.float32)
    neg_inf = jnp.full((LANES,), -jnp.inf, jnp.float32)
    bufs = [xa_v, xb_v]
    base = wid * ROWS_PER_TILE

    def nfill(j, carry):
        out_v[pl.ds(j * LANES, LANES)] = neg_inf
        return carry
    lax.fori_loop(0, NVREG, nfill, 0, unroll=16)
    xa_v[pl.ds(N, LANES)] = neg_inf
    xb_v[pl.ds(N, LANES)] = neg_inf

    def _solve(x_v, r):
        kvb = kv_v.at[r % 2]
        kib = ki_v.at[r % 2]

        def fill(j, carry):
            ci_v[pl.ds(j * LANES, LANES)] = jnp.full((LANES,), N, jnp.int32)
            return carry
        lax.fori_loop(0, NV_CAND, fill, 0, unroll=NV_CAND)

        th = jnp.full((LANES,), T0, jnp.float32)

        def cbody(i, carry):
            off, iv = carry
            v = x_v[pl.ds(i * LANES, LANES)]
            mk = v >= th
            pos = plsc.cumsum(mk.astype(jnp.int32))
            idx = jnp.minimum(off + pos, jnp.int32(CAP_PAD - 1))
            plsc.store_scatter(ci_v, [idx], iv, mask=mk)
            return (off + plsc.all_reduce_population_count(mk),
                    iv + LANES)

        off, _ = lax.fori_loop(0, NVREG, cbody, (zeros - 1, lane),
                               unroll=16)
        n = off[0] + 1

        def mbody(j, carry):
            iv = ci_v[pl.ds(j * LANES, LANES)]
            cs_v[pl.ds(j * LANES, LANES)] = plsc.load_gather(x_v, [iv])
            return carry
        lax.fori_loop(0, NV_CAND, mbody, 0, unroll=NV_CAND)

        def fast_branch():
            def count_ge(cand):
                def gbody(j, acc):
                    v = cs_v[pl.ds(j * LANES, LANES)]
                    sk = lax.bitcast_convert_type(v, jnp.int32)
                    return acc + plsc.all_reduce_population_count(sk >= cand)
                return lax.fori_loop(0, NV_CAND, gbody, zeros,
                                     unroll=NV_CAND)

            t = _search_t(count_ge, K)

            def gebody(j, carry):
                g, e = carry
                v = cs_v[pl.ds(j * LANES, LANES)]
                sk = lax.bitcast_convert_type(v, jnp.int32)
                g = g + plsc.all_reduce_population_count(sk > t)
                e = e + plsc.all_reduce_population_count(sk == t)
                return (g, e)

            cnt_gt, cnt_eq = lax.fori_loop(0, NV_CAND, gebody,
                                           (zeros, zeros), unroll=NV_CAND)
            need = K - cnt_gt

            def no_tie():
                return jnp.full((LANES,), N - 1, jnp.int32)

            def tie():
                def tie_body(i, pm):
                    bit = jnp.left_shift(jnp.int32(1), jnp.int32(14) - i)
                    cand = pm & ~bit

                    def tbody(j, acc):
                        v = cs_v[pl.ds(j * LANES, LANES)]
                        sk = lax.bitcast_convert_type(v, jnp.int32)
                        iv = ci_v[pl.ds(j * LANES, LANES)]
                        mm = (sk == t) & (iv <= cand)
                        return acc + plsc.all_reduce_population_count(mm)

                    f = lax.fori_loop(0, NV_CAND, tbody, zeros)
                    return jnp.where(f >= need, cand, pm)

                return lax.fori_loop(0, 15, tie_body,
                                     jnp.full((LANES,), N - 1, jnp.int32))

            m = lax.cond(cnt_eq[0] == need[0], no_tie, tie)

            def ebody(j, koff):
                v = cs_v[pl.ds(j * LANES, LANES)]
                sk = lax.bitcast_convert_type(v, jnp.int32)
                iv = ci_v[pl.ds(j * LANES, LANES)]
                mk = (sk > t) | ((sk == t) & (iv <= m))
                plsc.store_compressed(kvb.at[pl.ds(koff, LANES)], v, mask=mk)
                plsc.store_compressed(kib.at[pl.ds(koff, LANES)], iv, mask=mk)
                return koff + plsc.all_reduce_population_count(mk)[0]

            lax.fori_loop(0, NV_CAND, ebody, jnp.int32(0), unroll=NV_CAND)

        def slow_branch():
            def kbody(i, carry):
                v = x_v[pl.ds(i * LANES, LANES)]
                x_v[pl.ds(i * LANES, LANES)] = lax.bitcast_convert_type(
                    _keys(v), jnp.float32)
                return carry
            lax.fori_loop(0, NVREG, kbody, 0, unroll=8)

            def count_ge(cand):
                def gbody(j, acc):
                    v = x_v[pl.ds(j * LANES, LANES)]
                    sk = lax.bitcast_convert_type(v, jnp.int32)
                    return acc + plsc.all_reduce_population_count(sk >= cand)
                return lax.fori_loop(0, NVREG, gbody, zeros)

            t = _search_t(count_ge, K)

            def gebody(j, carry):
                g, e = carry
                v = x_v[pl.ds(j * LANES, LANES)]
                sk = lax.bitcast_convert_type(v, jnp.int32)
                g = g + plsc.all_reduce_population_count(sk > t)
                e = e + plsc.all_reduce_population_count(sk == t)
                return (g, e)

            cnt_gt, _ = lax.fori_loop(0, NVREG, gebody, (zeros, zeros))
            need = K - cnt_gt

            def tie_body(i, pm):
                bit = jnp.left_shift(jnp.int32(1), jnp.int32(14) - i)
                cand = pm & ~bit

                def tbody(j, carry):
                    acc, iv = carry
                    v = x_v[pl.ds(j * LANES, LANES)]
                    sk = lax.bitcast_convert_type(v, jnp.int32)
                    mm = (sk == t) & (iv <= cand)
                    return (acc + plsc.all_reduce_population_count(mm),
                            iv + LANES)

                f, _ = lax.fori_loop(0, NVREG, tbody, (zeros, lane))
                return jnp.where(f >= need, cand, pm)

            m = lax.fori_loop(0, 15, tie_body,
                              jnp.full((LANES,), N - 1, jnp.int32))

            def ebody(j, carry):
                koff, iv = carry
                kv = x_v[pl.ds(j * LANES, LANES)]
                sk = lax.bitcast_convert_type(kv, jnp.int32)
                mk = (sk > t) | ((sk == t) & (iv <= m))
                plsc.store_compressed(kvb.at[pl.ds(koff, LANES)],
                                      _inv_keys(sk), mask=mk)
                plsc.store_compressed(kib.at[pl.ds(koff, LANES)],
                                      iv, mask=mk)
                return (koff + plsc.all_reduce_population_count(mk)[0],
                        iv + LANES)

            lax.fori_loop(0, NVREG, ebody, (jnp.int32(0), lane))

        lax.cond((n >= K) & (n <= CAP_FAST), fast_branch, slow_branch)

    in_h = [None] * ROWS_PER_TILE
    out_h = None
    in_h[0] = pltpu.make_async_copy(
        x_hbm.at[base], bufs[0].at[pl.ds(0, N)], isem.at[0])
    in_h[0].start()
    for r in range(ROWS_PER_TILE):
        b = r % 2
        if r + 1 < ROWS_PER_TILE:
            in_h[r + 1] = pltpu.make_async_copy(
                x_hbm.at[base + r + 1], bufs[(r + 1) % 2].at[pl.ds(0, N)],
                isem.at[(r + 1) % 2])
            in_h[r + 1].start()
        in_h[r].wait()
        _solve(bufs[b], r)
        if r >= 1:
            out_h.wait()
            kprev = ki_v.at[(r - 1) % 2]
            for l in range(K // LANES):
                pidx = kprev[pl.ds(l * LANES, LANES)]
                plsc.store_scatter(out_v, [pidx], neg_inf)
        kvb = kv_v.at[b]
        kib = ki_v.at[b]
        for l in range(K // LANES):
            kidx = kib[pl.ds(l * LANES, LANES)]
            kval = kvb[pl.ds(l * LANES, LANES)]
            plsc.store_scatter(out_v, [kidx], kval)
        out_h = pltpu.make_async_copy(
            out_v, out_hbm.at[base + r], osem.at[0])
        out_h.start()
    out_h.wait()


def kernel(x):
    mesh = plsc.VectorSubcoreMesh(core_axis_name="c", subcore_axis_name="s")
    f = pl.kernel(
        _sc_body,
        mesh=mesh,
        out_type=jax.ShapeDtypeStruct((ROWS, N), jnp.float32),
        compiler_params=pltpu.CompilerParams(needs_layout_passes=False),
        scratch_types=[
            pltpu.VMEM((N + LANES,), jnp.float32),
            pltpu.VMEM((N + LANES,), jnp.float32),
            pltpu.VMEM((CAP_PAD,), jnp.float32),
            pltpu.VMEM((CAP_PAD,), jnp.int32),
            pltpu.VMEM((2, K), jnp.float32),
            pltpu.VMEM((2, K), jnp.int32),
            pltpu.VMEM((N,), jnp.float32),
            pltpu.SemaphoreType.DMA((2,)),
            pltpu.SemaphoreType.DMA((1,)),
        ],
    )
    return f(x)


# speedup vs baseline: 1.1966x; 1.1966x over previous
import jax
import jax.numpy as jnp
from jax import lax
from jax.experimental import pallas as pl
from jax.experimental.pallas import tpu as pltpu
from jax.experimental.pallas import tpu_sc as plsc

K = 64
ROWS = 128
N = 32768
LANES = 16
NVREG = N // LANES
ROWS_PER_TILE = ROWS // 32
T0 = 2.5
CAP_FAST = 320
CAP_PAD = CAP_FAST + LANES
NV_CAND = CAP_PAD // LANES
INT_MIN = -2147483648
MASK_POS = 0x7FFFFFFF


def _keys(v):
    xi = lax.bitcast_convert_type(v, jnp.int32)
    return jnp.where(xi >= 0, xi, jnp.bitwise_xor(xi, jnp.int32(MASK_POS)))


def _inv_keys(sk):
    xi = jnp.where(sk >= 0, sk, jnp.bitwise_xor(sk, jnp.int32(MASK_POS)))
    return lax.bitcast_convert_type(xi, jnp.float32)


def _search_t(count_ge, k):
    def sbody(i, p):
        bit = jnp.left_shift(jnp.int32(1), jnp.int32(31) - i)
        cand = p + bit
        return jnp.where(count_ge(cand) >= k, cand, p)
    return lax.fori_loop(0, 32, sbody, jnp.full((LANES,), INT_MIN, jnp.int32))


def _sc_body(x_hbm, out_hbm, xa_v, xb_v, cs_v, ci_v, kv_v, ki_v,
             out_v, isem, osem):
    c = lax.axis_index("c")
    s_ax = lax.axis_index("s")
    wid = s_ax * 2 + c
    lane = lax.iota(jnp.int32, LANES)
    zeros = jnp.zeros((LANES,), jnp.int32)
    sent_f = lax.bitcast_convert_type(
        jnp.full((LANES,), INT_MIN, jnp.int32), jnp.float32)
    neg_inf = jnp.full((LANES,), -jnp.inf, jnp.float32)
    bufs = [xa_v, xb_v]
    base = wid * ROWS_PER_TILE

    def nfill(j, carry):
        out_v[pl.ds(j * LANES, LANES)] = neg_inf
        return carry
    lax.fori_loop(0, NVREG, nfill, 0, unroll=16)
    xa_v[pl.ds(N, LANES)] = neg_inf
    xb_v[pl.ds(N, LANES)] = neg_inf

    def _solve(x_v, r):
        kvb = kv_v.at[r % 2]
        kib = ki_v.at[r % 2]

        def fill(j, carry):
            ci_v[pl.ds(j * LANES, LANES)] = jnp.full((LANES,), N, jnp.int32)
            return carry
        lax.fori_loop(0, NV_CAND, fill, 0, unroll=NV_CAND)

        th = jnp.full((LANES,), T0, jnp.float32)

        def cbody(i, carry):
            ntot, off, iv = carry
            v = x_v[pl.ds(i * LANES, LANES)]
            mk = v >= th
            plsc.store_compressed(ci_v.at[pl.ds(off, LANES)], iv, mask=mk)
            pc = plsc.all_reduce_population_count(mk)[0]
            return (ntot + pc,
                    jnp.minimum(off + pc, jnp.int32(CAP_FAST)),
                    iv + LANES)

        n, _, _ = lax.fori_loop(0, NVREG, cbody,
                                (jnp.int32(0), jnp.int32(0), lane),
                                unroll=16)

        def mbody(j, carry):
            iv = ci_v[pl.ds(j * LANES, LANES)]
            cs_v[pl.ds(j * LANES, LANES)] = plsc.load_gather(x_v, [iv])
            return carry
        lax.fori_loop(0, NV_CAND, mbody, 0, unroll=NV_CAND)

        def fast_branch():
            def count_ge(cand):
                def gbody(j, acc):
                    v = cs_v[pl.ds(j * LANES, LANES)]
                    sk = lax.bitcast_convert_type(v, jnp.int32)
                    return acc + plsc.all_reduce_population_count(sk >= cand)
                return lax.fori_loop(0, NV_CAND, gbody, zeros,
                                     unroll=NV_CAND)

            t = _search_t(count_ge, K)

            def gebody(j, carry):
                g, e = carry
                v = cs_v[pl.ds(j * LANES, LANES)]
                sk = lax.bitcast_convert_type(v, jnp.int32)
                g = g + plsc.all_reduce_population_count(sk > t)
                e = e + plsc.all_reduce_population_count(sk == t)
                return (g, e)

            cnt_gt, cnt_eq = lax.fori_loop(0, NV_CAND, gebody,
                                           (zeros, zeros), unroll=NV_CAND)
            need = K - cnt_gt

            def no_tie():
                return jnp.full((LANES,), N - 1, jnp.int32)

            def tie():
                def tie_body(i, pm):
                    bit = jnp.left_shift(jnp.int32(1), jnp.int32(14) - i)
                    cand = pm & ~bit

                    def tbody(j, acc):
                        v = cs_v[pl.ds(j * LANES, LANES)]
                        sk = lax.bitcast_convert_type(v, jnp.int32)
                        iv = ci_v[pl.ds(j * LANES, LANES)]
                        mm = (sk == t) & (iv <= cand)
                        return acc + plsc.all_reduce_population_count(mm)

                    f = lax.fori_loop(0, NV_CAND, tbody, zeros)
                    return jnp.where(f >= need, cand, pm)

                return lax.fori_loop(0, 15, tie_body,
                                     jnp.full((LANES,), N - 1, jnp.int32))

            m = lax.cond(cnt_eq[0] == need[0], no_tie, tie)

            def ebody(j, koff):
                v = cs_v[pl.ds(j * LANES, LANES)]
                sk = lax.bitcast_convert_type(v, jnp.int32)
                iv = ci_v[pl.ds(j * LANES, LANES)]
                mk = (sk > t) | ((sk == t) & (iv <= m))
                plsc.store_compressed(kvb.at[pl.ds(koff, LANES)], v, mask=mk)
                plsc.store_compressed(kib.at[pl.ds(koff, LANES)], iv, mask=mk)
                return koff + plsc.all_reduce_population_count(mk)[0]

            lax.fori_loop(0, NV_CAND, ebody, jnp.int32(0), unroll=NV_CAND)

        def slow_branch():
            def kbody(i, carry):
                v = x_v[pl.ds(i * LANES, LANES)]
                x_v[pl.ds(i * LANES, LANES)] = lax.bitcast_convert_type(
                    _keys(v), jnp.float32)
                return carry
            lax.fori_loop(0, NVREG, kbody, 0, unroll=8)

            def count_ge(cand):
                def gbody(j, acc):
                    v = x_v[pl.ds(j * LANES, LANES)]
                    sk = lax.bitcast_convert_type(v, jnp.int32)
                    return acc + plsc.all_reduce_population_count(sk >= cand)
                return lax.fori_loop(0, NVREG, gbody, zeros)

            t = _search_t(count_ge, K)

            def gebody(j, carry):
                g, e = carry
                v = x_v[pl.ds(j * LANES, LANES)]
                sk = lax.bitcast_convert_type(v, jnp.int32)
                g = g + plsc.all_reduce_population_count(sk > t)
                e = e + plsc.all_reduce_population_count(sk == t)
                return (g, e)

            cnt_gt, _ = lax.fori_loop(0, NVREG, gebody, (zeros, zeros))
            need = K - cnt_gt

            def tie_body(i, pm):
                bit = jnp.left_shift(jnp.int32(1), jnp.int32(14) - i)
                cand = pm & ~bit

                def tbody(j, carry):
                    acc, iv = carry
                    v = x_v[pl.ds(j * LANES, LANES)]
                    sk = lax.bitcast_convert_type(v, jnp.int32)
                    mm = (sk == t) & (iv <= cand)
                    return (acc + plsc.all_reduce_population_count(mm),
                            iv + LANES)

                f, _ = lax.fori_loop(0, NVREG, tbody, (zeros, lane))
                return jnp.where(f >= need, cand, pm)

            m = lax.fori_loop(0, 15, tie_body,
                              jnp.full((LANES,), N - 1, jnp.int32))

            def ebody(j, carry):
                koff, iv = carry
                kv = x_v[pl.ds(j * LANES, LANES)]
                sk = lax.bitcast_convert_type(kv, jnp.int32)
                mk = (sk > t) | ((sk == t) & (iv <= m))
                plsc.store_compressed(kvb.at[pl.ds(koff, LANES)],
                                      _inv_keys(sk), mask=mk)
                plsc.store_compressed(kib.at[pl.ds(koff, LANES)],
                                      iv, mask=mk)
                return (koff + plsc.all_reduce_population_count(mk)[0],
                        iv + LANES)

            lax.fori_loop(0, NVREG, ebody, (jnp.int32(0), lane))

        lax.cond((n >= K) & (n <= CAP_FAST), fast_branch, slow_branch)

    in_h = [None] * ROWS_PER_TILE
    out_h = None
    in_h[0] = pltpu.make_async_copy(
        x_hbm.at[base], bufs[0].at[pl.ds(0, N)], isem.at[0])
    in_h[0].start()
    for r in range(ROWS_PER_TILE):
        b = r % 2
        if r + 1 < ROWS_PER_TILE:
            in_h[r + 1] = pltpu.make_async_copy(
                x_hbm.at[base + r + 1], bufs[(r + 1) % 2].at[pl.ds(0, N)],
                isem.at[(r + 1) % 2])
            in_h[r + 1].start()
        in_h[r].wait()
        _solve(bufs[b], r)
        if r >= 1:
            out_h.wait()
            kprev = ki_v.at[(r - 1) % 2]
            for l in range(K // LANES):
                pidx = kprev[pl.ds(l * LANES, LANES)]
                plsc.store_scatter(out_v, [pidx], neg_inf)
        kvb = kv_v.at[b]
        kib = ki_v.at[b]
        for l in range(K // LANES):
            kidx = kib[pl.ds(l * LANES, LANES)]
            kval = kvb[pl.ds(l * LANES, LANES)]
            plsc.store_scatter(out_v, [kidx], kval)
        out_h = pltpu.make_async_copy(
            out_v, out_hbm.at[base + r], osem.at[0])
        out_h.start()
    out_h.wait()


def kernel(x):
    mesh = plsc.VectorSubcoreMesh(core_axis_name="c", subcore_axis_name="s")
    f = pl.kernel(
        _sc_body,
        mesh=mesh,
        out_type=jax.ShapeDtypeStruct((ROWS, N), jnp.float32),
        compiler_params=pltpu.CompilerParams(needs_layout_passes=False),
        scratch_types=[
            pltpu.VMEM((N + LANES,), jnp.float32),
            pltpu.VMEM((N + LANES,), jnp.float32),
            pltpu.VMEM((CAP_PAD,), jnp.float32),
            pltpu.VMEM((CAP_PAD,), jnp.int32),
            pltpu.VMEM((2, K), jnp.float32),
            pltpu.VMEM((2, K), jnp.int32),
            pltpu.VMEM((N,), jnp.float32),
            pltpu.SemaphoreType.DMA((2,)),
            pltpu.SemaphoreType.DMA((1,)),
        ],
    )
    return f(x)
